# baseline (speedup 1.0000x reference)
import functools

import jax
import jax.numpy as jnp
from jax import lax
from jax.experimental import pallas as pl
from jax.experimental.pallas import tpu as pltpu
from jax.experimental.pallas import tpu_sc as plsc

_NC, _NS = 2, 16
_NW = _NC * _NS
_CH = 80


_NB = 5


def _make_sc_agg(n_nodes, n_edges, d, split_d=False):
    pack = 128 // d
    ew = n_edges // (_NS if split_d else _NW)
    assert ew % (_CH * _NB) == 0 and _CH % pack == 0
    n_chunks = ew // _CH
    n_groups = n_chunks // _NB
    erows = _CH // pack
    du = d // 16
    rows_pt = (n_nodes // _NS) & ~7
    rows_last = n_nodes - rows_pt * (_NS - 1)
    zr = 208
    nz_full = rows_pt // zr
    assert nz_full * zr == rows_pt
    zrem_last = rows_last - rows_pt
    assert 0 < zrem_last <= zr and zrem_last % 8 == 0
    nd16 = d // 16
    mesh = plsc.VectorSubcoreMesh(core_axis_name="c", subcore_axis_name="s")

    @functools.partial(
        pl.kernel,
        out_type=jax.ShapeDtypeStruct((_NC, n_nodes, d), jnp.float32),
        mesh=mesh,
        scratch_types=(
            [pltpu.VMEM((_CH,), jnp.int32) for _ in range(_NB)]
            + [pltpu.VMEM((_CH,), jnp.int32) for _ in range(_NB)]
            + [
                pltpu.VMEM((_NB, erows, 128), jnp.float32),
                pltpu.VMEM((_NB, _CH, d), jnp.float32),
                pltpu.VMEM((zr, d), jnp.float32),
                pltpu.VMEM_SHARED((n_nodes, d), jnp.float32),
                pltpu.SemaphoreType.DMA,
                pltpu.SemaphoreType.DMA,
            ]
        ),
        compiler_params=pltpu.CompilerParams(use_tc_tiling_on_sc=False),
    )
    def agg_kernel(table, src, dst, e, out, *scr):
        idxs_l = scr[:_NB]
        idxd_l = scr[_NB:2 * _NB]
        m_v, g_v, z_v, acc_sh, semf, semg = scr[2 * _NB:]
        cid = lax.axis_index("c")
        sid = lax.axis_index("s")
        wid = sid * _NC + cid

        def zrow(r, carry):
            for j in range(nd16):
                z_v[r, pl.ds(j * 16, 16)] = jnp.zeros((16,), jnp.float32)
            return carry
        lax.fori_loop(0, zr, zrow, 0)
        base_r = sid * rows_pt
        for i in range(nz_full):
            pltpu.sync_copy(z_v, acc_sh.at[pl.ds(base_r + i * zr, zr)])

        @pl.when(sid == _NS - 1)
        def _():
            pltpu.sync_copy(z_v.at[pl.ds(0, zrem_last)],
                            acc_sh.at[pl.ds(base_r + rows_pt, zrem_last)])
        plsc.subcore_barrier()

        if split_d:
            ibase = sid * ew
            ebase = (cid * n_edges + ibase) // pack
            idx_off = cid * n_nodes
        else:
            ibase = wid * ew
            ebase = ibase // pack
            idx_off = 0

        def fetch_descs(c, b):
            ib = pl.multiple_of(ibase + c * _CH, 8)
            eb = ebase + c * erows
            return (
                pltpu.make_async_copy(src.at[pl.ds(ib, _CH)], idxs_l[b], semf),
                pltpu.make_async_copy(dst.at[pl.ds(ib, _CH)], idxd_l[b], semf),
                pltpu.make_async_copy(e.at[pl.ds(eb, erows)], m_v.at[b], semf),
            )

        def gather_desc(b):
            return pltpu.make_async_copy(table.at[idxs_l[b]], g_v.at[b], semg)

        for b in range(_NB):
            for dsc in fetch_descs(b, b):
                dsc.start()

        def group(g, carry):
            c0 = g * _NB
            gds = []
            for b in range(_NB):
                for dsc in fetch_descs(c0 + b, b):
                    dsc.wait()
                if split_d:
                    for j in range(_CH // 16):
                        sl = pl.ds(j * 16, 16)
                        idxs_l[b][sl] = idxs_l[b][sl] + idx_off
                gd = gather_desc(b)
                gd.start()
                gds.append(gd)
            for b in range(_NB):
                c = c0 + b
                gds[b].wait()

                def crow(r, c2):
                    for j in range(du):
                        u = r * du + j
                        mr = lax.shift_right_logical(u, 3)
                        mc = lax.mul(lax.rem(u, 8), 16)
                        s = pl.ds(j * 16, 16)
                        g_v[b, r, s] = jnp.maximum(
                            g_v[b, r, s] + m_v[b, mr, pl.ds(mc, 16)], 0.0)
                    return c2
                lax.fori_loop(0, _CH, crow, 0)
                pltpu.sync_copy(g_v.at[b], acc_sh.at[idxd_l[b]], add=True)

                @pl.when(c + _NB < n_chunks)
                def _():
                    for dsc in fetch_descs(c + _NB, b):
                        dsc.start()
            return carry
        lax.fori_loop(0, n_groups, group, 0)

        plsc.subcore_barrier()
        for i in range(nz_full):
            rb = base_r + i * zr
            pltpu.sync_copy(acc_sh.at[pl.ds(rb, zr)], out.at[cid, pl.ds(rb, zr)])

        @pl.when(sid == _NS - 1)
        def _():
            rb = base_r + rows_pt
            pltpu.sync_copy(acc_sh.at[pl.ds(rb, zrem_last)],
                            out.at[cid, pl.ds(rb, zrem_last)])

    return agg_kernel


def _edge_proj1(ea2, W1bd, b1bd):
    rows, k2 = ea2.shape
    blk = 3200
    nb = rows // blk

    def body(a_ref, w_ref, b_ref, e_ref):
        e_ref[:] = (jnp.dot(a_ref[:], w_ref[0],
                            preferred_element_type=jnp.float32) + b_ref[0, 0:1, :])

    return pl.pallas_call(
        body,
        grid=(2, nb),
        in_specs=[
            pl.BlockSpec((blk, k2), lambda h, i: (i, 0)),
            pl.BlockSpec((1, k2, 128), lambda h, i: (h, 0, 0)),
            pl.BlockSpec((1, 8, 128), lambda h, i: (h, 0, 0)),
        ],
        out_specs=pl.BlockSpec((blk, 128), lambda h, i: (h * nb + i, 0)),
        out_shape=jax.ShapeDtypeStruct((2 * rows, 128), jnp.float32),
    )(ea2, W1bd, b1bd)


def _edge_proj23(ea4, W2bd, b2bd, W3bd, b3bd):
    rows, k4 = ea4.shape
    blk = 1600
    nb = rows // blk

    def body(a_ref, w2_ref, b2_ref, w3_ref, b3_ref, e2_ref, e3_ref):
        a = a_ref[:]
        e2_ref[:] = jnp.dot(a, w2_ref[:], preferred_element_type=jnp.float32) + b2_ref[0:1, :]
        e3_ref[:] = jnp.dot(a, w3_ref[:], preferred_element_type=jnp.float32) + b3_ref[0:1, :]

    return pl.pallas_call(
        body,
        grid=(nb,),
        in_specs=[
            pl.BlockSpec((blk, k4), lambda i: (i, 0)),
            pl.BlockSpec((k4, 128), lambda i: (0, 0)),
            pl.BlockSpec((8, 128), lambda i: (0, 0)),
            pl.BlockSpec((k4, 128), lambda i: (0, 0)),
            pl.BlockSpec((8, 128), lambda i: (0, 0)),
        ],
        out_specs=[
            pl.BlockSpec((blk, 128), lambda i: (i, 0)),
            pl.BlockSpec((blk, 128), lambda i: (i, 0)),
        ],
        out_shape=[
            jax.ShapeDtypeStruct((rows, 128), jnp.float32),
            jax.ShapeDtypeStruct((rows, 128), jnp.float32),
        ],
    )(ea4, W2bd, b2bd, W3bd, b3bd)


def _node_update(h, agg, W, b):
    n = h.shape[0]
    dim_halves = agg.shape[2] != h.shape[1]

    def body(h_ref, a_ref, w_ref, b_ref, o_ref):
        if dim_halves:
            s = h_ref[:] + jnp.concatenate([a_ref[0], a_ref[1]], axis=1)
        else:
            s = h_ref[:] + a_ref[0] + a_ref[1]
        o_ref[:] = jnp.maximum(
            jnp.dot(s, w_ref[:], preferred_element_type=jnp.float32) + b_ref[:], 0.0)

    return pl.pallas_call(
        body,
        out_shape=jax.ShapeDtypeStruct((n, W.shape[1]), jnp.float32),
    )(h, agg, W, b)


def _final(h2, agg3, W3, b3, batch2d, Wlp, blp, n_graphs):
    n, dd = h2.shape
    ncls = Wlp.shape[1]

    def body(h_ref, a_ref, w_ref, b_ref, bat_ref, wl_ref, bl_ref, o_ref,
             h3_ref, mx_ref):
        s = h_ref[:] + a_ref[0] + a_ref[1]
        h3_ref[:] = jnp.maximum(
            jnp.dot(s, w_ref[:], preferred_element_type=jnp.float32) + b_ref[:], 0.0)
        bat = bat_ref[:]
        h3 = h3_ref[:]
        onehot = (bat == lax.broadcasted_iota(jnp.int32, (n, n_graphs), 1)
                  ).astype(jnp.float32)
        sums = lax.dot_general(onehot, h3, (((0,), (0,)), ((), ())),
                               preferred_element_type=jnp.float32)
        counts = jnp.sum(onehot, axis=0)[:, None]
        mean = sums / jnp.maximum(counts, 1.0)

        def gbody(g, carry):
            mask = bat == g
            hm = jnp.where(mask, h3, -jnp.inf)
            mx_ref[pl.ds(g, 1), :] = jnp.max(hm, axis=0, keepdims=True)
            return carry
        lax.fori_loop(0, n_graphs, gbody, 0)
        mxv = mx_ref[:]
        mxv = jnp.where(jnp.isfinite(mxv), mxv, 0.0)
        pooled = jnp.concatenate([mxv, mean], axis=1)
        o_ref[:] = jnp.dot(pooled, wl_ref[:], preferred_element_type=jnp.float32) + bl_ref[:]

    return pl.pallas_call(
        body,
        out_shape=jax.ShapeDtypeStruct((n_graphs, ncls), jnp.float32),
        scratch_shapes=[
            pltpu.VMEM((n, dd), jnp.float32),
            pltpu.VMEM((n_graphs, dd), jnp.float32),
        ],
    )(h2, agg3, W3, b3, batch2d, Wlp, blp)


def kernel(x, edge_index, edge_attr, batch, W1, b1, We1, be1, W2, b2, We2, be2,
           W3, b3, We3, be3, Wlin, blin):
    n, d = x.shape
    n_edges = edge_index.shape[1]
    hid = W1.shape[1]
    hp = 32
    ncls = Wlin.shape[1]
    n_graphs = 64
    src = edge_index[0]
    dst = edge_index[1]

    def padw(a, rows, cols):
        return jnp.zeros((rows, cols), jnp.float32).at[:a.shape[0], :a.shape[1]].set(a)

    def padb(b_, cols, rows=1):
        return jnp.zeros((rows, cols), jnp.float32).at[:, :b_.shape[0]].set(b_[None, :])

    W1p = padw(W1, d, hp)
    b1p = padb(b1, hp)
    W2p = padw(W2, hp, hp)
    b2p = padb(b2, hp)
    W3p = padw(W3, hp, hp)
    b3p = padb(b3, hp)
    Wlp = (jnp.zeros((2 * hp, ncls), jnp.float32)
           .at[:hid].set(Wlin[:hid])
           .at[hp:hp + hid].set(Wlin[hid:]))
    blp = blin[None, :]

    k = We1.shape[0]
    dh = d // 2
    W1bd = jnp.zeros((2, 2 * k, 2 * dh), jnp.float32)
    b1bd = jnp.zeros((2, 8, 2 * dh), jnp.float32)
    for h in range(2):
        wh = lax.dynamic_slice_in_dim(We1, h * dh, dh, axis=1)
        bh = lax.dynamic_slice_in_dim(be1, h * dh, dh)
        W1bd = W1bd.at[h, :k, :dh].set(wh).at[h, k:, dh:].set(wh)
        b1bd = b1bd.at[h].set(jnp.tile(jnp.concatenate([bh, bh])[None, :], (8, 1)))

    def bd4(We_, be_):
        wp = padw(We_, k, hp)
        bp = jnp.zeros((hp,), jnp.float32).at[:be_.shape[0]].set(be_)
        W = jnp.zeros((4 * k, 4 * hp), jnp.float32)
        for j in range(4):
            W = W.at[j * k:(j + 1) * k, j * hp:(j + 1) * hp].set(wp)
        bt = jnp.tile(jnp.concatenate([bp] * 4)[None, :], (8, 1))
        return W, bt

    W2bd, b2bd = bd4(We2, be2)
    W3bd, b3bd = bd4(We3, be3)

    ea2 = edge_attr.reshape(n_edges // 2, 2 * k)
    ea4 = edge_attr.reshape(n_edges // 4, 4 * k)
    e1 = _edge_proj1(ea2, W1bd, b1bd)
    e2, e3 = _edge_proj23(ea4, W2bd, b2bd, W3bd, b3bd)

    x_cat = jnp.concatenate([x[:, :dh], x[:, dh:]], axis=0)
    agg_d = _make_sc_agg(n, n_edges, dh, split_d=True)
    agg_h = _make_sc_agg(n, n_edges, hp)

    a1 = agg_d(x_cat, src, dst, e1)
    h1 = _node_update(x, a1, W1p, b1p)
    a2 = agg_h(h1, src, dst, e2)
    h2 = _node_update(h1, a2, W2p, b2p)
    a3 = agg_h(h2, src, dst, e3)
    return _final(h2, a3, W3p, b3p, batch[:, None], Wlp, blp, n_graphs)


# speedup vs baseline: 1.5538x; 1.5538x over previous
import functools

import jax
import jax.numpy as jnp
from jax import lax
from jax.experimental import pallas as pl
from jax.experimental.pallas import tpu as pltpu
from jax.experimental.pallas import tpu_sc as plsc

_NC, _NS = 2, 16
_NW = _NC * _NS
_CH = 80


_NB = 5


def _make_sc_agg(n_nodes, n_edges, d, split_d=False):
    pack = 128 // d
    ew = n_edges // (_NS if split_d else _NW)
    assert ew % (_CH * _NB) == 0 and _CH % pack == 0
    n_chunks = ew // _CH
    n_groups = n_chunks // _NB
    erows = _CH // pack
    du = d // 16
    rows_pt = (n_nodes // _NS) & ~7
    rows_last = n_nodes - rows_pt * (_NS - 1)
    zr = 208
    nz_full = rows_pt // zr
    assert nz_full * zr == rows_pt
    zrem_last = rows_last - rows_pt
    assert 0 < zrem_last <= zr and zrem_last % 8 == 0
    nd16 = d // 16
    mesh = plsc.VectorSubcoreMesh(core_axis_name="c", subcore_axis_name="s")

    @functools.partial(
        pl.kernel,
        out_type=jax.ShapeDtypeStruct((_NC, n_nodes, d), jnp.float32),
        mesh=mesh,
        scratch_types=(
            [pltpu.VMEM((_CH,), jnp.int32) for _ in range(_NB)]
            + [pltpu.VMEM((_CH,), jnp.int32) for _ in range(_NB)]
            + [
                pltpu.VMEM((_NB, erows, 128), jnp.float32),
                pltpu.VMEM((_NB, _CH, d), jnp.float32),
                pltpu.VMEM((zr, d), jnp.float32),
                pltpu.VMEM_SHARED((n_nodes, d), jnp.float32),
                pltpu.SemaphoreType.DMA,
                pltpu.SemaphoreType.DMA,
            ]
        ),
        compiler_params=pltpu.CompilerParams(use_tc_tiling_on_sc=False),
    )
    def agg_kernel(table, src, dst, e, out, *scr):
        idxs_l = scr[:_NB]
        idxd_l = scr[_NB:2 * _NB]
        m_v, g_v, z_v, acc_sh, semf, semg = scr[2 * _NB:]
        cid = lax.axis_index("c")
        sid = lax.axis_index("s")
        wid = sid * _NC + cid

        def zrow(r, carry):
            for j in range(nd16):
                z_v[r, pl.ds(j * 16, 16)] = jnp.zeros((16,), jnp.float32)
            return carry
        lax.fori_loop(0, zr, zrow, 0)
        base_r = sid * rows_pt
        for i in range(nz_full):
            pltpu.sync_copy(z_v, acc_sh.at[pl.ds(base_r + i * zr, zr)])

        @pl.when(sid == _NS - 1)
        def _():
            pltpu.sync_copy(z_v.at[pl.ds(0, zrem_last)],
                            acc_sh.at[pl.ds(base_r + rows_pt, zrem_last)])
        plsc.subcore_barrier()

        if split_d:
            ibase = sid * ew
            ebase = (cid * n_edges + ibase) // pack
            idx_off = cid * n_nodes
        else:
            ibase = wid * ew
            ebase = ibase // pack
            idx_off = 0

        def fetch_descs(c, b):
            ib = pl.multiple_of(ibase + c * _CH, 8)
            eb = ebase + c * erows
            return (
                pltpu.make_async_copy(src.at[pl.ds(ib, _CH)], idxs_l[b], semf),
                pltpu.make_async_copy(dst.at[pl.ds(ib, _CH)], idxd_l[b], semf),
                pltpu.make_async_copy(e.at[pl.ds(eb, erows)], m_v.at[b], semf),
            )

        def gather_desc(b):
            return pltpu.make_async_copy(table.at[idxs_l[b]], g_v.at[b], semg)

        for b in range(_NB):
            for dsc in fetch_descs(b, b):
                dsc.start()

        def group(g, carry):
            c0 = g * _NB
            gds = []
            for b in range(_NB):
                for dsc in fetch_descs(c0 + b, b):
                    dsc.wait()
                if split_d:
                    for j in range(_CH // 16):
                        sl = pl.ds(j * 16, 16)
                        idxs_l[b][sl] = idxs_l[b][sl] + idx_off
                gd = gather_desc(b)
                gd.start()
                gds.append(gd)
            for b in range(_NB):
                c = c0 + b
                gds[b].wait()

                def crow(pr, c2):
                    for q in range(pack):
                        r = pr * pack + q
                        for j in range(du):
                            g_v[b, r, pl.ds(j * 16, 16)] = jnp.maximum(
                                g_v[b, r, pl.ds(j * 16, 16)]
                                + m_v[b, pr, pl.ds((q * du + j) * 16, 16)], 0.0)
                    return c2
                lax.fori_loop(0, erows, crow, 0)
                pltpu.sync_copy(g_v.at[b], acc_sh.at[idxd_l[b]], add=True)

                @pl.when(c + _NB < n_chunks)
                def _():
                    for dsc in fetch_descs(c + _NB, b):
                        dsc.start()
            return carry
        lax.fori_loop(0, n_groups, group, 0)

        plsc.subcore_barrier()
        for i in range(nz_full):
            rb = base_r + i * zr
            pltpu.sync_copy(acc_sh.at[pl.ds(rb, zr)], out.at[cid, pl.ds(rb, zr)])

        @pl.when(sid == _NS - 1)
        def _():
            rb = base_r + rows_pt
            pltpu.sync_copy(acc_sh.at[pl.ds(rb, zrem_last)],
                            out.at[cid, pl.ds(rb, zrem_last)])

    return agg_kernel


def _edge_proj1(ea, W1bd, b1bd):
    rows, k2 = ea.shape
    blk = 3200
    nb = rows // blk

    def body(a_ref, w_ref, b_ref, e_ref):
        e_ref[:] = (jnp.dot(a_ref[:], w_ref[0],
                            preferred_element_type=jnp.float32) + b_ref[0, 0:1, :])

    return pl.pallas_call(
        body,
        grid=(2, nb),
        in_specs=[
            pl.BlockSpec((blk, k2), lambda h, i: (i, 0)),
            pl.BlockSpec((1, k2, 128), lambda h, i: (h, 0, 0)),
            pl.BlockSpec((1, 8, 128), lambda h, i: (h, 0, 0)),
        ],
        out_specs=pl.BlockSpec((blk, 128), lambda h, i: (h * nb + i, 0)),
        out_shape=jax.ShapeDtypeStruct((2 * rows, 128), jnp.float32),
    )(ea, W1bd, b1bd)


def _edge_proj23(ea, W2bd, b2bd, W3bd, b3bd):
    rows, k4 = ea.shape
    blk = 1600
    nb = rows // blk

    def body(a_ref, w2_ref, b2_ref, w3_ref, b3_ref, e2_ref, e3_ref):
        a = a_ref[:]
        e2_ref[:] = jnp.dot(a, w2_ref[:], preferred_element_type=jnp.float32) + b2_ref[0:1, :]
        e3_ref[:] = jnp.dot(a, w3_ref[:], preferred_element_type=jnp.float32) + b3_ref[0:1, :]

    return pl.pallas_call(
        body,
        grid=(nb,),
        in_specs=[
            pl.BlockSpec((blk, k4), lambda i: (i, 0)),
            pl.BlockSpec((k4, 128), lambda i: (0, 0)),
            pl.BlockSpec((8, 128), lambda i: (0, 0)),
            pl.BlockSpec((k4, 128), lambda i: (0, 0)),
            pl.BlockSpec((8, 128), lambda i: (0, 0)),
        ],
        out_specs=[
            pl.BlockSpec((blk, 128), lambda i: (i, 0)),
            pl.BlockSpec((blk, 128), lambda i: (i, 0)),
        ],
        out_shape=[
            jax.ShapeDtypeStruct((rows, 128), jnp.float32),
            jax.ShapeDtypeStruct((rows, 128), jnp.float32),
        ],
    )(ea, W2bd, b2bd, W3bd, b3bd)


def _node_update(h, agg, W, b):
    n = h.shape[0]
    dim_halves = agg.shape[2] != h.shape[1]

    def body(h_ref, a_ref, w_ref, b_ref, o_ref):
        if dim_halves:
            s = h_ref[:] + jnp.concatenate([a_ref[0], a_ref[1]], axis=1)
        else:
            s = h_ref[:] + a_ref[0] + a_ref[1]
        o_ref[:] = jnp.maximum(
            jnp.dot(s, w_ref[:], preferred_element_type=jnp.float32) + b_ref[:], 0.0)

    return pl.pallas_call(
        body,
        out_shape=jax.ShapeDtypeStruct((n, W.shape[1]), jnp.float32),
    )(h, agg, W, b)


def _final(h2, agg3, W3, b3, batch2d, Wlp, blp, n_graphs):
    n, dd = h2.shape
    ncls = Wlp.shape[1]

    def body(h_ref, a_ref, w_ref, b_ref, bat_ref, wl_ref, bl_ref, o_ref,
             va_ref, vb_ref):
        s = h_ref[:] + a_ref[0] + a_ref[1]
        va_ref[:] = jnp.maximum(
            jnp.dot(s, w_ref[:], preferred_element_type=jnp.float32) + b_ref[:], 0.0)
        bat = bat_ref[:]
        h3 = va_ref[:]
        onehot = (bat == lax.broadcasted_iota(jnp.int32, (n, n_graphs), 1)
                  ).astype(jnp.float32)
        sums = lax.dot_general(onehot, h3, (((0,), (0,)), ((), ())),
                               preferred_element_type=jnp.float32)
        counts = jnp.sum(onehot, axis=0)[:, None]
        mean = sums / jnp.maximum(counts, 1.0)

        def gbody(g, carry):
            mask = bat == g
            hm = jnp.where(mask, h3, -jnp.inf)
            vb_ref[pl.ds(g, 1), :] = jnp.max(hm, axis=0, keepdims=True)
            return carry
        lax.fori_loop(0, n_graphs, gbody, 0)
        mxv = vb_ref[0:n_graphs, :]
        mxv = jnp.where(jnp.isfinite(mxv), mxv, 0.0)
        pooled = jnp.concatenate([mxv, mean], axis=1)
        o_ref[:] = jnp.dot(pooled, wl_ref[:], preferred_element_type=jnp.float32) + bl_ref[:]

    return pl.pallas_call(
        body,
        out_shape=jax.ShapeDtypeStruct((n_graphs, ncls), jnp.float32),
        scratch_shapes=[
            pltpu.VMEM((n, dd), jnp.float32),
            pltpu.VMEM((n_graphs, dd), jnp.float32),
        ],
    )(h2, agg3, W3, b3, batch2d, Wlp, blp)


def kernel(x, edge_index, edge_attr, batch, W1, b1, We1, be1, W2, b2, We2, be2,
           W3, b3, We3, be3, Wlin, blin):
    n, d = x.shape
    n_edges = edge_index.shape[1]
    hid = W1.shape[1]
    hp = 32
    ncls = Wlin.shape[1]
    n_graphs = 64
    src = edge_index[0]
    dst = edge_index[1]

    def padw(a, rows, cols):
        return jnp.zeros((rows, cols), jnp.float32).at[:a.shape[0], :a.shape[1]].set(a)

    def padb(b_, cols, rows=1):
        return jnp.zeros((rows, cols), jnp.float32).at[:, :b_.shape[0]].set(b_[None, :])

    W1p = padw(W1, d, hp)
    b1p = padb(b1, hp)
    W2p = padw(W2, hp, hp)
    b2p = padb(b2, hp)
    W3p = padw(W3, hp, hp)
    b3p = padb(b3, hp)
    Wlp = (jnp.zeros((2 * hp, ncls), jnp.float32)
           .at[:hid].set(Wlin[:hid])
           .at[hp:hp + hid].set(Wlin[hid:]))
    blp = blin[None, :]

    k = We1.shape[0]
    dh = d // 2
    W1bd = jnp.zeros((2, 2 * k, 2 * dh), jnp.float32)
    b1bd = jnp.zeros((2, 8, 2 * dh), jnp.float32)
    for h in range(2):
        wh = lax.dynamic_slice_in_dim(We1, h * dh, dh, axis=1)
        bh = lax.dynamic_slice_in_dim(be1, h * dh, dh)
        W1bd = W1bd.at[h, :k, :dh].set(wh).at[h, k:, dh:].set(wh)
        b1bd = b1bd.at[h].set(jnp.tile(jnp.concatenate([bh, bh])[None, :], (8, 1)))

    def bd4(We_, be_):
        wp = padw(We_, k, hp)
        bp = jnp.zeros((hp,), jnp.float32).at[:be_.shape[0]].set(be_)
        W = jnp.zeros((4 * k, 4 * hp), jnp.float32)
        for j in range(4):
            W = W.at[j * k:(j + 1) * k, j * hp:(j + 1) * hp].set(wp)
        bt = jnp.tile(jnp.concatenate([bp] * 4)[None, :], (8, 1))
        return W, bt

    W2bd, b2bd = bd4(We2, be2)
    W3bd, b3bd = bd4(We3, be3)

    ea2 = edge_attr.reshape(n_edges // 2, 2 * k)
    ea4 = edge_attr.reshape(n_edges // 4, 4 * k)
    e1 = _edge_proj1(ea2, W1bd, b1bd)
    e2, e3 = _edge_proj23(ea4, W2bd, b2bd, W3bd, b3bd)

    x_cat = jnp.concatenate([x[:, :dh], x[:, dh:]], axis=0)
    agg_d = _make_sc_agg(n, n_edges, dh, split_d=True)
    agg_h = _make_sc_agg(n, n_edges, hp)

    a1 = agg_d(x_cat, src, dst, e1)
    h1 = _node_update(x, a1, W1p, b1p)
    a2 = agg_h(h1, src, dst, e2)
    h2 = _node_update(h1, a2, W2p, b2p)
    a3 = agg_h(h2, src, dst, e3)
    return _final(h2, a3, W3p, b3p, batch[:, None], Wlp, blp, n_graphs)
